# baseline (speedup 1.0000x reference)
import jax, jax.numpy as jnp
from jax.experimental import pallas as pl
from jax.experimental.pallas import tpu as pltpu

def _body(lat_ref, out_ref):
    out_ref[...] = lat_ref[...]

def kernel(latents, msg, W_emb):
    B, C, H, W = latents.shape
    S = C * 8
    lat = latents.reshape(B, S, 128)
    f = pl.pallas_call(
        _body,
        grid=(B,),
        in_specs=[pl.BlockSpec((1, S, 128), lambda b: (b, 0, 0))],
        out_specs=pl.BlockSpec((1, S, 128), lambda b: (b, 0, 0)),
        out_shape=jax.ShapeDtypeStruct((B, S, 128), jnp.float32),
        compiler_params=pltpu.CompilerParams(
            dimension_semantics=("arbitrary",)),
    )
    return f(lat).reshape(B, C, H, W)


# speedup vs baseline: 1.9884x; 1.9884x over previous
import functools

import jax
import jax.numpy as jnp
from jax import lax
from jax.experimental import pallas as pl
from jax.experimental.pallas import tpu as pltpu
from jax.experimental.pallas import tpu_sc as plsc

_LANES = 16


@functools.lru_cache(maxsize=None)
def _make_sc_kernel(B, C, SP, L):
    HALF = C // 2
    CH = _LANES
    NK = HALF // CH
    NSEG = SP // _LANES
    mesh = plsc.VectorSubcoreMesh(core_axis_name="c", subcore_axis_name="s")

    @functools.partial(
        pl.kernel,
        out_type=jax.ShapeDtypeStruct((B, 2 * C, SP), jnp.float32),
        mesh=mesh,
        scratch_types=[
            pltpu.VMEM((L,), jnp.int32),
            pltpu.VMEM((L, C), jnp.float32),
            pltpu.VMEM((C,), jnp.float32),
            pltpu.VMEM((3, CH, SP), jnp.float32),
            pltpu.VMEM((2, CH, SP), jnp.float32),
            pltpu.SemaphoreType.DMA,
            pltpu.SemaphoreType.DMA((3,)),
            pltpu.SemaphoreType.DMA((3,)),
            pltpu.SemaphoreType.DMA((2,)),
        ],
    )
    def sc_kernel(idx_hbm, w_hbm, lat_hbm, out_hbm,
                  idx_v, rows_v, aux_v, lbuf, cbuf,
                  gsem, isems, osems, csems):
        wid = lax.axis_index("s") * 2 + lax.axis_index("c")
        b = wid // 2
        c0 = (wid % 2) * HALF

        pltpu.sync_copy(idx_hbm.at[b], idx_v)
        pltpu.async_copy(w_hbm.at[idx_v], rows_v, gsem).wait()

        def asum(j, carry):
            sl = pl.ds(j * _LANES, _LANES)
            acc = rows_v[0, sl]
            for l in range(1, L):
                acc = acc + rows_v[l, sl]
            aux_v[sl] = acc
            return carry

        lax.fori_loop(0, C // _LANES, asum, 0)

        def in_b(k):
            return pltpu.make_async_copy(
                lat_hbm.at[b, pl.ds(c0 + k * CH, CH)],
                lbuf.at[k % 3], isems.at[k % 3])

        def out_b(k):
            return pltpu.make_async_copy(
                lbuf.at[k % 3],
                out_hbm.at[b, pl.ds(c0 + k * CH, CH)], osems.at[k % 3])

        def out_c(k):
            return pltpu.make_async_copy(
                cbuf.at[k % 2],
                out_hbm.at[b, pl.ds(C + c0 + k * CH, CH)], csems.at[k % 2])

        def fill_c(k):
            bi = k % 2
            grp = aux_v[pl.ds(c0 + k * CH, CH)]
            for j in range(CH):
                v = jnp.full((_LANES,), grp[j], jnp.float32)
                for g in range(NSEG):
                    cbuf[bi, j, pl.ds(g * _LANES, _LANES)] = v

        def chunk(k, head):
            if head < 2:
                if head >= 1:
                    pass
            in_b(k).wait()
            out_b(k).start()
            fill_c(k)
            out_c(k).start()

        in_b(0).start()
        in_b(1).start()
        chunk(0, 0)
        in_b(2).start()
        chunk(1, 1)

        def body(k, carry):
            out_b(k - 2).wait()
            out_c(k - 2).wait()
            in_b(k + 1).start()
            in_b(k).wait()
            out_b(k).start()
            fill_c(k)
            out_c(k).start()
            return carry

        lax.fori_loop(2, NK - 1, body, 0)

        out_b(NK - 3).wait()
        out_c(NK - 3).wait()
        in_b(NK - 1).wait()
        out_b(NK - 1).start()
        fill_c(NK - 1)
        out_c(NK - 1).start()

        out_b(NK - 2).wait()
        out_c(NK - 2).wait()
        out_b(NK - 1).wait()
        out_c(NK - 1).wait()

    return sc_kernel


def kernel(latents, msg, W_emb):
    B, C, H, W = latents.shape
    L = msg.shape[-1]
    msg_i = msg.astype(jnp.int32)
    idx = (2 * jnp.arange(L, dtype=jnp.int32))[None, :] + msg_i
    out = _make_sc_kernel(B, C, H * W, L)(
        idx, W_emb, latents.reshape(B, C, H * W))
    return out.reshape(B, 2 * C, H, W)
